# baseline (speedup 1.0000x reference)
import functools

import jax
import jax.numpy as jnp
from jax import lax
from jax.experimental import pallas as pl
from jax.experimental.pallas import tpu as pltpu
from jax.experimental.pallas import tpu_sc as plsc

N = 10000
E = 320000
H = 128
G = 64
OUT = 128

BN = 2000
GRID = N // BN

NC = 2
NS = 16
NW = NC * NS
EPW = E // NW
CHUNK = 50
NCHUNK = EPW // CHUNK

F32 = jnp.float32


_sc_mesh = plsc.VectorSubcoreMesh(core_axis_name="c", subcore_axis_name="s")

NBUF = 4
GLEAD = 3

CHUNK = 64
ROWS_BASE = 78
CH_BASE = 2 * ROWS_BASE


@functools.partial(
    pl.kernel,
    mesh=_sc_mesh,
    out_type=jax.ShapeDtypeStruct((NC, N, H), F32),
    scratch_types=[
        pltpu.VMEM((NBUF, CHUNK), jnp.int32),
        pltpu.VMEM((NBUF, CHUNK), jnp.int32),
        pltpu.VMEM((NBUF, CHUNK, H), F32),
        pltpu.VMEM_SHARED((N, H), F32),
    ] + [pltpu.SemaphoreType.DMA] * (2 * NBUF),
)
def _edge_agg(m_hbm, src_hbm, dst_hbm, zero_hbm, out_hbm,
              sidx, didx, rows, accum, *sems):
    isem = sems[:NBUF]
    gsem = sems[NBUF:]
    c = lax.axis_index("c")
    s = lax.axis_index("s")
    wid = s * NC + c
    base = (wid * ROWS_BASE + jnp.minimum(wid, 4)) * 128
    cnt = CH_BASE + 2 * (wid < 4).astype(jnp.int32)

    def fire_idx(k, t):
        pltpu.async_copy(src_hbm.at[pl.ds(base + k * CHUNK, CHUNK)],
                         sidx.at[t], isem[t])
        pltpu.async_copy(dst_hbm.at[pl.ds(base + k * CHUNK, CHUNK)],
                         didx.at[t], isem[t])

    def wait_idx(k, t):
        pltpu.make_async_copy(src_hbm.at[pl.ds(base + k * CHUNK, CHUNK)],
                              sidx.at[t], isem[t]).wait()
        pltpu.make_async_copy(dst_hbm.at[pl.ds(base + k * CHUNK, CHUNK)],
                              didx.at[t], isem[t]).wait()

    def fire_gather(t):
        pltpu.async_copy(m_hbm.at[sidx.at[t]], rows.at[t], gsem[t])

    def wait_gather(t):
        pltpu.make_async_copy(m_hbm.at[sidx.at[t]], rows.at[t],
                              gsem[t]).wait()

    for t in range(NBUF):
        fire_idx(t, t)
    for t in range(GLEAD):
        wait_idx(t, t)
        fire_gather(t)

    @pl.when(s < 10)
    def _():
        pltpu.sync_copy(zero_hbm, accum.at[pl.ds(s * 1000, 1000)])

    plsc.subcore_barrier()

    def slot(k, t):
        tg = (t + GLEAD) % NBUF

        @pl.when(k + GLEAD < cnt)
        def _():
            wait_idx(k + GLEAD, tg)
            fire_gather(tg)

        wait_gather(t)
        pltpu.sync_copy(rows.at[t], accum.at[didx.at[t]], add=True)

        @pl.when(k + NBUF < cnt)
        def _():
            fire_idx(k + NBUF, t)

    def body(j, carry):
        for t in range(NBUF):
            slot(j * NBUF + t, t)
        return carry

    lax.fori_loop(0, CH_BASE // NBUF, body, 0)

    for k in range(CH_BASE, CH_BASE + 2):
        @pl.when(cnt > k)
        def _():
            slot(k, k % NBUF)

    plsc.subcore_barrier()

    @pl.when(s < 10)
    def _():
        pltpu.sync_copy(accum.at[pl.ds(s * 1000, 1000)],
                        out_hbm.at[c, pl.ds(s * 1000, 1000)])


EDGE_ROWS_PAD = 2560
_EB = 16384


def _relay_body(e_ref, so_ref, do_ref):
    eb = e_ref[...]
    so_ref[...] = eb[0].reshape(128, 128)
    do_ref[...] = eb[1].reshape(128, 128)


_relay_call = pl.pallas_call(
    _relay_body,
    grid=(EDGE_ROWS_PAD // 128,),
    in_specs=[
        pl.BlockSpec((2, _EB), lambda i: (0, i)),
    ],
    out_specs=[
        pl.BlockSpec((128, 128), lambda i: (i, 0)),
        pl.BlockSpec((128, 128), lambda i: (i, 0)),
    ],
    out_shape=[
        jax.ShapeDtypeStruct((EDGE_ROWS_PAD, 128), jnp.int32),
        jax.ShapeDtypeStruct((EDGE_ROWS_PAD, 128), jnp.int32),
    ],
)


BF16 = jnp.bfloat16


def _gru(parts0, parts1, gh, h, w_ih_t, b_ih):
    agg = (parts0 + parts1).astype(BF16)
    gi = jnp.dot(agg, w_ih_t, preferred_element_type=F32) + b_ih
    r = jax.nn.sigmoid(gi[:, :H] + gh[:, :H])
    z = jax.nn.sigmoid(gi[:, H:2 * H] + gh[:, H:2 * H])
    n = jnp.tanh(gi[:, 2 * H:] + r * gh[:, 2 * H:])
    return (1.0 - z) * n + z * h


def _entry_body(x_ref, w_in_ref, b_in_ref, g_ref, w_hh_ref, b_hh_ref,
                h_ref, m_ref, gh_ref):
    xb = x_ref[...].astype(BF16)
    h = jnp.dot(xb, w_in_ref[...], preferred_element_type=F32) + b_in_ref[...]
    h_ref[...] = h
    hb = h.astype(BF16)
    m_ref[...] = jnp.dot(hb, g_ref[...], preferred_element_type=F32)
    gh_ref[...] = (jnp.dot(hb, w_hh_ref[...], preferred_element_type=F32)
                   + b_hh_ref[...]).astype(BF16)


def _mid_body(parts_ref, gh_ref, h_ref, w_ih_ref, b_ih_ref, g_ref,
              w_hh_ref, b_hh_ref, h_out_ref, m_ref, gh_out_ref):
    h_new = _gru(parts_ref[0], parts_ref[1], gh_ref[...].astype(F32),
                 h_ref[...], w_ih_ref[...], b_ih_ref[...])
    h_out_ref[...] = h_new
    hb = h_new.astype(BF16)
    m_ref[...] = jnp.dot(hb, g_ref[...], preferred_element_type=F32)
    gh_out_ref[...] = (jnp.dot(hb, w_hh_ref[...], preferred_element_type=F32)
                       + b_hh_ref[...]).astype(BF16)


def _final_body(parts_ref, gh_ref, h_ref, w_ih_ref, b_ih_ref, batch_ref,
                w_out_ref, b_out_ref, out_ref, sums_ref, cnt_ref):
    i = pl.program_id(0)

    @pl.when(i == 0)
    def _():
        sums_ref[...] = jnp.zeros_like(sums_ref)
        cnt_ref[...] = jnp.zeros_like(cnt_ref)

    h_new = _gru(parts_ref[0], parts_ref[1], gh_ref[...].astype(F32),
                 h_ref[...], w_ih_ref[...], b_ih_ref[...])
    b_blk = batch_ref[0, 0, :]
    oh = (b_blk[:, None] == lax.broadcasted_iota(jnp.int32, (BN, G), 1)).astype(F32)
    sums_ref[...] += lax.dot_general(oh, h_new, (((0,), (0,)), ((), ())),
                                     preferred_element_type=F32)
    cnt_ref[...] += lax.dot_general(oh, jnp.ones((BN, 8), F32),
                                    (((0,), (0,)), ((), ())),
                                    preferred_element_type=F32)

    @pl.when(i == GRID - 1)
    def _():
        pooled = sums_ref[...] / jnp.maximum(cnt_ref[...][:, :1], 1.0)
        out_ref[...] = jnp.dot(pooled, w_out_ref[...],
                               preferred_element_type=F32) + b_out_ref[...]


def _row_spec(width):
    return pl.BlockSpec((BN, width), lambda i: (i, 0))


def _full_spec(rows, cols):
    return pl.BlockSpec((rows, cols), lambda i: (0, 0))


_entry_call = pl.pallas_call(
    _entry_body,
    grid=(GRID,),
    in_specs=[
        _row_spec(H),
        _full_spec(H, H),
        _full_spec(1, H),
        _full_spec(H, H),
        _full_spec(H, 3 * H),
        _full_spec(1, 3 * H),
    ],
    out_specs=[_row_spec(H), _row_spec(H), _row_spec(3 * H)],
    out_shape=[
        jax.ShapeDtypeStruct((N, H), F32),
        jax.ShapeDtypeStruct((N, H), F32),
        jax.ShapeDtypeStruct((N, 3 * H), BF16),
    ],
)

_mid_call = pl.pallas_call(
    _mid_body,
    grid=(GRID,),
    in_specs=[
        pl.BlockSpec((NC, BN, H), lambda i: (0, i, 0)),
        _row_spec(3 * H),
        _row_spec(H),
        _full_spec(H, 3 * H),
        _full_spec(1, 3 * H),
        _full_spec(H, H),
        _full_spec(H, 3 * H),
        _full_spec(1, 3 * H),
    ],
    out_specs=[_row_spec(H), _row_spec(H), _row_spec(3 * H)],
    out_shape=[
        jax.ShapeDtypeStruct((N, H), F32),
        jax.ShapeDtypeStruct((N, H), F32),
        jax.ShapeDtypeStruct((N, 3 * H), BF16),
    ],
)

_final_call = pl.pallas_call(
    _final_body,
    grid=(GRID,),
    in_specs=[
        pl.BlockSpec((NC, BN, H), lambda i: (0, i, 0)),
        _row_spec(3 * H),
        _row_spec(H),
        _full_spec(H, 3 * H),
        _full_spec(1, 3 * H),
        pl.BlockSpec((1, 1, BN), lambda i: (i, 0, 0)),
        _full_spec(H, OUT),
        _full_spec(1, OUT),
    ],
    out_specs=pl.BlockSpec((G, OUT), lambda i: (0, 0)),
    out_shape=jax.ShapeDtypeStruct((G, OUT), F32),
    scratch_shapes=[
        pltpu.VMEM((G, OUT), F32),
        pltpu.VMEM((G, 8), F32),
    ],
)


def kernel(node_embed, edge_index, batch, W_in, b_in, ggnn_w, W_ih, W_hh,
           b_ih, b_hh, W_out, b_out):
    src2, dst2 = _relay_call(edge_index)
    src2 = src2.reshape(EDGE_ROWS_PAD * 128)
    dst2 = dst2.reshape(EDGE_ROWS_PAD * 128)
    w_in_t = W_in.T.astype(BF16)
    w_ih_t = W_ih.T.astype(BF16)
    w_hh_t = W_hh.T.astype(BF16)
    w_out_t = W_out.T
    ggnn_w = ggnn_w.astype(BF16)
    b_in2 = b_in.reshape(1, H)
    b_ih2 = b_ih.reshape(1, 3 * H)
    b_hh2 = b_hh.reshape(1, 3 * H)
    b_out2 = b_out.reshape(1, OUT)
    batch3 = batch.reshape(GRID, 1, BN)
    zeros = jnp.zeros((1000, H), F32)

    h, m, gh = _entry_call(node_embed, w_in_t, b_in2, ggnn_w[0], w_hh_t, b_hh2)
    out = None
    for layer in range(3):
        parts = _edge_agg(m, src2, dst2, zeros)
        if layer < 2:
            h, m, gh = _mid_call(parts, gh, h, w_ih_t, b_ih2,
                                 ggnn_w[layer + 1], w_hh_t, b_hh2)
        else:
            out = _final_call(parts, gh, h, w_ih_t, b_ih2, batch3,
                              w_out_t, b_out2)
    return out


# speedup vs baseline: 1.3775x; 1.3775x over previous
import functools

import jax
import jax.numpy as jnp
from jax import lax
from jax.experimental import pallas as pl
from jax.experimental.pallas import tpu as pltpu
from jax.experimental.pallas import tpu_sc as plsc

N = 10000
E = 320000
H = 128
G = 64
OUT = 128

BN = 2000
GRID = N // BN

NC = 2
NS = 16
NW = NC * NS
EPW = E // NW
CHUNK = 50
NCHUNK = EPW // CHUNK

F32 = jnp.float32


_sc_mesh = plsc.VectorSubcoreMesh(core_axis_name="c", subcore_axis_name="s")

NBUF = 4
GLEAD = 2

CHUNK = 64
ROWS_BASE = 78
CH_BASE = 2 * ROWS_BASE


@functools.partial(
    pl.kernel,
    mesh=_sc_mesh,
    out_type=jax.ShapeDtypeStruct((NC, N, H), F32),
    scratch_types=[
        pltpu.VMEM((NBUF, CHUNK), jnp.int32),
        pltpu.VMEM((NBUF, CHUNK), jnp.int32),
        pltpu.VMEM((NBUF, CHUNK, H), F32),
        pltpu.VMEM_SHARED((N, H), F32),
    ] + [pltpu.SemaphoreType.DMA] * (2 * NBUF),
)
def _edge_agg(m_hbm, src_hbm, dst_hbm, zero_hbm, out_hbm,
              sidx, didx, rows, accum, *sems):
    isem = sems[:NBUF]
    gsem = sems[NBUF:]
    c = lax.axis_index("c")
    s = lax.axis_index("s")
    wid = s * NC + c
    base = (wid * ROWS_BASE + jnp.minimum(wid, 4)) * 128
    cnt = CH_BASE + 2 * (wid < 4).astype(jnp.int32)

    def fire_idx(k, t):
        pltpu.async_copy(src_hbm.at[pl.ds(base + k * CHUNK, CHUNK)],
                         sidx.at[t], isem[t])
        pltpu.async_copy(dst_hbm.at[pl.ds(base + k * CHUNK, CHUNK)],
                         didx.at[t], isem[t])

    def wait_idx(k, t):
        pltpu.make_async_copy(src_hbm.at[pl.ds(base + k * CHUNK, CHUNK)],
                              sidx.at[t], isem[t]).wait()
        pltpu.make_async_copy(dst_hbm.at[pl.ds(base + k * CHUNK, CHUNK)],
                              didx.at[t], isem[t]).wait()

    def fire_gather(t):
        pltpu.async_copy(m_hbm.at[sidx.at[t]], rows.at[t], gsem[t])

    def wait_gather(t):
        pltpu.make_async_copy(m_hbm.at[sidx.at[t]], rows.at[t],
                              gsem[t]).wait()

    for t in range(NBUF):
        fire_idx(t, t)
    for t in range(GLEAD):
        wait_idx(t, t)
        fire_gather(t)

    @pl.when(s < 10)
    def _():
        pltpu.sync_copy(zero_hbm, accum.at[pl.ds(s * 1000, 1000)])

    plsc.subcore_barrier()

    def slot(k, t):
        tg = (t + GLEAD) % NBUF

        @pl.when(k + GLEAD < cnt)
        def _():
            wait_idx(k + GLEAD, tg)
            fire_gather(tg)

        wait_gather(t)
        pltpu.sync_copy(rows.at[t], accum.at[didx.at[t]], add=True)

        @pl.when(k + NBUF < cnt)
        def _():
            fire_idx(k + NBUF, t)

    def body(j, carry):
        for t in range(NBUF):
            slot(j * NBUF + t, t)
        return carry

    lax.fori_loop(0, CH_BASE // NBUF, body, 0)

    for k in range(CH_BASE, CH_BASE + 2):
        @pl.when(cnt > k)
        def _():
            slot(k, k % NBUF)

    plsc.subcore_barrier()

    @pl.when(s < 10)
    def _():
        pltpu.sync_copy(accum.at[pl.ds(s * 1000, 1000)],
                        out_hbm.at[c, pl.ds(s * 1000, 1000)])


EDGE_ROWS_PAD = 2560
_EB = 16384


def _relay_body(e_ref, so_ref, do_ref):
    eb = e_ref[...]
    so_ref[...] = eb[0].reshape(128, 128)
    do_ref[...] = eb[1].reshape(128, 128)


_relay_call = pl.pallas_call(
    _relay_body,
    grid=(EDGE_ROWS_PAD // 128,),
    in_specs=[
        pl.BlockSpec((2, _EB), lambda i: (0, i)),
    ],
    out_specs=[
        pl.BlockSpec((128, 128), lambda i: (i, 0)),
        pl.BlockSpec((128, 128), lambda i: (i, 0)),
    ],
    out_shape=[
        jax.ShapeDtypeStruct((EDGE_ROWS_PAD, 128), jnp.int32),
        jax.ShapeDtypeStruct((EDGE_ROWS_PAD, 128), jnp.int32),
    ],
)


BF16 = jnp.bfloat16


def _gru(parts0, parts1, gh, h, w_ih_t, b_ih):
    agg = (parts0 + parts1).astype(BF16)
    gi = jnp.dot(agg, w_ih_t, preferred_element_type=F32) + b_ih
    r = jax.nn.sigmoid(gi[:, :H] + gh[:, :H])
    z = jax.nn.sigmoid(gi[:, H:2 * H] + gh[:, H:2 * H])
    n = jnp.tanh(gi[:, 2 * H:] + r * gh[:, 2 * H:])
    return (1.0 - z) * n + z * h


def _entry_body(x_ref, w_in_ref, b_in_ref, g_ref, w_hh_ref, b_hh_ref,
                h_ref, m_ref, gh_ref):
    xb = x_ref[...].astype(BF16)
    h = jnp.dot(xb, w_in_ref[...], preferred_element_type=F32) + b_in_ref[...]
    h_ref[...] = h.astype(BF16)
    hb = h.astype(BF16)
    m_ref[...] = jnp.dot(hb, g_ref[...], preferred_element_type=F32)
    gh_ref[...] = (jnp.dot(hb, w_hh_ref[...], preferred_element_type=F32)
                   + b_hh_ref[...]).astype(BF16)


def _mid_body(parts_ref, gh_ref, h_ref, w_ih_ref, b_ih_ref, g_ref,
              w_hh_ref, b_hh_ref, h_out_ref, m_ref, gh_out_ref):
    h_new = _gru(parts_ref[0], parts_ref[1], gh_ref[...].astype(F32),
                 h_ref[...].astype(F32), w_ih_ref[...], b_ih_ref[...])
    h_out_ref[...] = h_new.astype(BF16)
    hb = h_new.astype(BF16)
    m_ref[...] = jnp.dot(hb, g_ref[...], preferred_element_type=F32)
    gh_out_ref[...] = (jnp.dot(hb, w_hh_ref[...], preferred_element_type=F32)
                       + b_hh_ref[...]).astype(BF16)


def _final_body(parts_ref, gh_ref, h_ref, w_ih_ref, b_ih_ref, batch_ref,
                w_out_ref, b_out_ref, out_ref, sums_ref, cnt_ref):
    i = pl.program_id(0)

    @pl.when(i == 0)
    def _():
        sums_ref[...] = jnp.zeros_like(sums_ref)
        cnt_ref[...] = jnp.zeros_like(cnt_ref)

    h_new = _gru(parts_ref[0], parts_ref[1], gh_ref[...].astype(F32),
                 h_ref[...].astype(F32), w_ih_ref[...], b_ih_ref[...])
    b_blk = batch_ref[0, 0, :]
    oh = (b_blk[:, None] == lax.broadcasted_iota(jnp.int32, (BN, G), 1)).astype(F32)
    sums_ref[...] += lax.dot_general(oh, h_new, (((0,), (0,)), ((), ())),
                                     preferred_element_type=F32)
    cnt_ref[...] += lax.dot_general(oh, jnp.ones((BN, 8), F32),
                                    (((0,), (0,)), ((), ())),
                                    preferred_element_type=F32)

    @pl.when(i == GRID - 1)
    def _():
        pooled = sums_ref[...] / jnp.maximum(cnt_ref[...][:, :1], 1.0)
        out_ref[...] = jnp.dot(pooled, w_out_ref[...],
                               preferred_element_type=F32) + b_out_ref[...]


def _row_spec(width):
    return pl.BlockSpec((BN, width), lambda i: (i, 0))


def _full_spec(rows, cols):
    return pl.BlockSpec((rows, cols), lambda i: (0, 0))


_entry_call = pl.pallas_call(
    _entry_body,
    grid=(GRID,),
    in_specs=[
        _row_spec(H),
        _full_spec(H, H),
        _full_spec(1, H),
        _full_spec(H, H),
        _full_spec(H, 3 * H),
        _full_spec(1, 3 * H),
    ],
    out_specs=[_row_spec(H), _row_spec(H), _row_spec(3 * H)],
    out_shape=[
        jax.ShapeDtypeStruct((N, H), BF16),
        jax.ShapeDtypeStruct((N, H), F32),
        jax.ShapeDtypeStruct((N, 3 * H), BF16),
    ],
)

_mid_call = pl.pallas_call(
    _mid_body,
    grid=(GRID,),
    in_specs=[
        pl.BlockSpec((NC, BN, H), lambda i: (0, i, 0)),
        _row_spec(3 * H),
        _row_spec(H),
        _full_spec(H, 3 * H),
        _full_spec(1, 3 * H),
        _full_spec(H, H),
        _full_spec(H, 3 * H),
        _full_spec(1, 3 * H),
    ],
    out_specs=[_row_spec(H), _row_spec(H), _row_spec(3 * H)],
    out_shape=[
        jax.ShapeDtypeStruct((N, H), BF16),
        jax.ShapeDtypeStruct((N, H), F32),
        jax.ShapeDtypeStruct((N, 3 * H), BF16),
    ],
)

_final_call = pl.pallas_call(
    _final_body,
    grid=(GRID,),
    in_specs=[
        pl.BlockSpec((NC, BN, H), lambda i: (0, i, 0)),
        _row_spec(3 * H),
        _row_spec(H),
        _full_spec(H, 3 * H),
        _full_spec(1, 3 * H),
        pl.BlockSpec((1, 1, BN), lambda i: (i, 0, 0)),
        _full_spec(H, OUT),
        _full_spec(1, OUT),
    ],
    out_specs=pl.BlockSpec((G, OUT), lambda i: (0, 0)),
    out_shape=jax.ShapeDtypeStruct((G, OUT), F32),
    scratch_shapes=[
        pltpu.VMEM((G, OUT), F32),
        pltpu.VMEM((G, 8), F32),
    ],
)


def kernel(node_embed, edge_index, batch, W_in, b_in, ggnn_w, W_ih, W_hh,
           b_ih, b_hh, W_out, b_out):
    src2, dst2 = _relay_call(edge_index)
    src2 = src2.reshape(EDGE_ROWS_PAD * 128)
    dst2 = dst2.reshape(EDGE_ROWS_PAD * 128)
    w_in_t = W_in.T.astype(BF16)
    w_ih_t = W_ih.T.astype(BF16)
    w_hh_t = W_hh.T.astype(BF16)
    w_out_t = W_out.T
    ggnn_w = ggnn_w.astype(BF16)
    b_in2 = b_in.reshape(1, H)
    b_ih2 = b_ih.reshape(1, 3 * H)
    b_hh2 = b_hh.reshape(1, 3 * H)
    b_out2 = b_out.reshape(1, OUT)
    batch3 = batch.reshape(GRID, 1, BN)
    zeros = jnp.zeros((1000, H), F32)

    h, m, gh = _entry_call(node_embed, w_in_t, b_in2, ggnn_w[0], w_hh_t, b_hh2)
    out = None
    for layer in range(3):
        parts = _edge_agg(m, src2, dst2, zeros)
        if layer < 2:
            h, m, gh = _mid_call(parts, gh, h, w_ih_t, b_ih2,
                                 ggnn_w[layer + 1], w_hh_t, b_hh2)
        else:
            out = _final_call(parts, gh, h, w_ih_t, b_ih2, batch3,
                              w_out_t, b_out2)
    return out
